# baseline (speedup 1.0000x reference)
import functools

import jax
import jax.numpy as jnp
from jax.experimental import pallas as pl
from jax.experimental.pallas import tpu as pltpu


def _ru(x, m):
    return (x + m - 1) // m * m


def _mm_act_kernel(x_ref, w_ref, b_ref, o_ref, *, act):
    y = jnp.dot(x_ref[...], w_ref[...], preferred_element_type=jnp.float32)
    y = y + b_ref[...]
    if act == "relu":
        y = jnp.maximum(y, 0.0)
    elif act == "sigmoid":
        y = jax.nn.sigmoid(y)
    o_ref[...] = y.astype(o_ref.dtype)


def _s2a_kernel(x_ref, w_ref, b_ref, o_ref, *, co, wo, tb, ck):
    xx = x_ref[0, 0].reshape(tb + 1, 2, wo + 1, 128)
    for c0 in range(0, tb, ck):
        wins = []
        for i in range(3):
            qi, si = divmod(i, 2)
            slab = xx[c0 + qi : c0 + qi + ck, si]
            wins.append(jnp.concatenate(
                [slab[:, 0:wo, :], slab[:, 1 : wo + 1, :]], axis=2))
        p = jnp.concatenate(wins, axis=2).reshape(ck * wo, 768)
        y = jnp.dot(p, w_ref[...], preferred_element_type=jnp.float32)
        y = jnp.maximum(y + b_ref[...], 0.0)
        o_ref[0, 0, c0 : c0 + ck] = y.reshape(ck, wo, co).astype(o_ref.dtype)


def _s2b_kernel(x_ref, w_ref, b_ref, o_ref, *, co, wo, tb, ck):
    xx = x_ref[0, 0].reshape(tb + 1, 2, wo + 1, 2, 128)
    for c0 in range(0, tb, ck):
        wins = []
        for i in range(3):
            qi, si = divmod(i, 2)
            slab = xx[c0 + qi : c0 + qi + ck, si]
            wins += [slab[:, 0:wo, 0, :], slab[:, 0:wo, 1, :],
                     slab[:, 1 : wo + 1, 0, :]]
        p = jnp.concatenate(wins, axis=2).reshape(ck * wo, 1152)
        y = jnp.dot(p, w_ref[...], preferred_element_type=jnp.float32)
        y = jnp.maximum(y + b_ref[...], 0.0)
        o_ref[0, 0, c0 : c0 + ck] = y.reshape(ck, wo, co).astype(o_ref.dtype)


def _ffc_fused_kernel(xl_ref, x_ref, wl_ref, wg_ref, b_ref, o_ref, *, tb, ck, wi):
    for c0 in range(0, tb, ck):
        xc = x_ref[0, 0, c0 : c0 + ck].reshape(ck * wi, 256)
        accg = jnp.dot(xc[:, 64:], wg_ref[...], preferred_element_type=jnp.float32)
        y1 = jnp.maximum(accg + b_ref[:, 256:], 0.0)
        accl = None
        for i in range(3):
            for j in range(3):
                slab = xl_ref[0, 0, c0 + i : c0 + i + ck, j : j + wi, :]
                d = jnp.dot(slab.reshape(ck * wi, 64), wl_ref[3 * i + j],
                            preferred_element_type=jnp.float32)
                accl = d if accl is None else accl + d
        y0 = jnp.maximum(accl + b_ref[:, :256], 0.0)
        o = y0 + y1 + xc.astype(jnp.float32)
        o_ref[0, 0, c0 : c0 + ck] = o.reshape(ck, wi, 256).astype(o_ref.dtype)


def _up_kernel(x_ref, w_ref, b_ref, o_ref, *, ci, co, wo, tb, ck):
    for c0 in range(0, tb, ck):
        acc = None
        for t, (i, j) in enumerate(((0, 0), (0, 1), (1, 0), (1, 1))):
            slab = x_ref[0, 0, c0 + i : c0 + i + ck, j : j + wo, :]
            d = jnp.dot(slab.reshape(ck * wo, ci), w_ref[t * ci : (t + 1) * ci],
                        preferred_element_type=jnp.float32)
            acc = d if acc is None else acc + d
        y = jnp.maximum(acc + b_ref[...], 0.0).astype(o_ref.dtype)
        y = y.reshape(ck, wo, 4 * co)
        for a in (0, 1):
            o_ref[0, 0, c0 : c0 + ck, a] = y[:, :, a * 2 * co : (a + 1) * 2 * co]


def _vmem_budget(*arrs):
    need = sum(2 * a.size * a.dtype.itemsize for a in arrs)
    return int(min(60 * (1 << 20), max(32 * (1 << 20), need)))


def _mm(x, w, b, act, tm, out_dtype=jnp.bfloat16, n_out=None):
    M, K = x.shape
    Kp, Np = w.shape
    Mp = _ru(M, tm)
    xb = x.astype(jnp.bfloat16)
    if (Mp, Kp) != (M, K):
        xb = jnp.pad(xb, ((0, Mp - M), (0, Kp - K)))
    out = pl.pallas_call(
        functools.partial(_mm_act_kernel, act=act),
        out_shape=jax.ShapeDtypeStruct((Mp, Np), out_dtype),
        grid=(Mp // tm,),
        in_specs=[
            pl.BlockSpec((tm, Kp), lambda i: (i, 0)),
            pl.BlockSpec((Kp, Np), lambda i: (0, 0)),
            pl.BlockSpec((1, Np), lambda i: (0, 0)),
        ],
        out_specs=pl.BlockSpec((tm, Np), lambda i: (i, 0)),
        compiler_params=pltpu.CompilerParams(
            dimension_semantics=("parallel",),
            vmem_limit_bytes=_vmem_budget(
                jax.ShapeDtypeStruct((tm, Kp), jnp.bfloat16),
                jax.ShapeDtypeStruct((tm, Np), out_dtype),
                jax.ShapeDtypeStruct((Kp, Np), jnp.bfloat16),
            ),
        ),
    )(xb, w, b)
    if n_out is None and Mp == M:
        return out
    return out[:M, : (Np if n_out is None else n_out)]


def _enc0_kernel(x_ref, w_ref, b_ref, o_ref, *, tb):
    wins = []
    for i in range(7):
        slab = x_ref[0, 0, i : i + tb]
        wins.append(jnp.concatenate([slab[:, 0:16, :], slab[:, 1:17, :]], axis=2))
    p = jnp.concatenate(wins, axis=2).reshape(tb * 16, 7 * 256)
    y = jnp.dot(p, w_ref[...], preferred_element_type=jnp.float32)
    y = jnp.maximum(y + b_ref[...], 0.0)
    o_ref[0, 0] = y.reshape(tb, 16, 2048).astype(o_ref.dtype)


def _enc0_conv(x, w, b, tb=8):
    N = x.shape[0]
    xp = jnp.pad(x, ((0, 0), (0, 0), (0, 26), (0, 0))).reshape(N, 518, 17, 128)
    nb = 512 // tb
    xb = _bands(xp, tb, tb + 6, nb)
    w7 = w[:196, :64].astype(jnp.float32).reshape(7, 7, 4, 64)
    wg = jnp.zeros((7, 64, 4, 32, 64), jnp.float32)
    for t in range(32):
        wg = wg.at[:, t : t + 7, :, t, :].set(w7)
    wg = wg.reshape(1792, 2048).astype(jnp.bfloat16)
    bg = jnp.tile(b[0, :64], 32).reshape(1, 2048)
    out = pl.pallas_call(
        functools.partial(_enc0_kernel, tb=tb),
        out_shape=jax.ShapeDtypeStruct((N, nb, tb, 16, 2048), jnp.bfloat16),
        grid=(N, nb),
        in_specs=[
            pl.BlockSpec((1, 1, tb + 6, 17, 128), lambda n, i: (n, i, 0, 0, 0)),
            pl.BlockSpec((1792, 2048), lambda n, i: (0, 0)),
            pl.BlockSpec((1, 2048), lambda n, i: (0, 0)),
        ],
        out_specs=pl.BlockSpec((1, 1, tb, 16, 2048), lambda n, i: (n, i, 0, 0, 0)),
        compiler_params=pltpu.CompilerParams(
            dimension_semantics=("parallel", "parallel"),
            vmem_limit_bytes=48 * (1 << 20),
        ),
    )(xb, wg, bg)
    return out.reshape(N, 512, 512, 64)


def _bands(x, row0_stride, rows_in, nb):
    return jnp.stack([x[:, row0_stride * b : row0_stride * b + rows_in]
                      for b in range(nb)], axis=1)


def _conv_s2(x, w, b, co, tb, ck):
    N, H, W, ci = x.shape
    wo = W // 2
    nb = (H // 2) // tb
    xp = jnp.pad(x, ((0, 0), (1, 1), (1, 1), (0, 0)))
    if ci == 64:
        xp = xp.reshape(N, H + 2, (W + 2) // 2, 128)
        xb = _bands(xp, 2 * tb, 2 * tb + 2, nb)
        wk = jnp.zeros((3, 4, 64, co), jnp.float32)
        wk = wk.at[:, :3].set(w[: 9 * ci, :co].astype(jnp.float32)
                              .reshape(3, 3, 64, co))
        wk = wk.reshape(768, co).astype(jnp.bfloat16)
        body = functools.partial(_s2a_kernel, co=co, wo=wo, tb=tb, ck=ck)
        in_shape = (1, 1, 2 * tb + 2, wo + 1, 128)
        wspec = pl.BlockSpec((768, co), lambda n, i: (0, 0))
    else:
        xb = _bands(xp, 2 * tb, 2 * tb + 2, nb)
        wk = w[: 9 * ci, :co]
        body = functools.partial(_s2b_kernel, co=co, wo=wo, tb=tb, ck=ck)
        in_shape = (1, 1, 2 * tb + 2, 2 * wo + 2, ci)
        wspec = pl.BlockSpec((9 * ci, co), lambda n, i: (0, 0))
    out = pl.pallas_call(
        body,
        out_shape=jax.ShapeDtypeStruct((N, nb, tb, wo, co), jnp.bfloat16),
        grid=(N, nb),
        in_specs=[
            pl.BlockSpec(in_shape, lambda n, i: (n, i, 0, 0, 0)),
            wspec,
            pl.BlockSpec((1, co), lambda n, i: (0, 0)),
        ],
        out_specs=pl.BlockSpec((1, 1, tb, wo, co), lambda n, i: (n, i, 0, 0, 0)),
        compiler_params=pltpu.CompilerParams(
            dimension_semantics=("parallel", "parallel"),
            vmem_limit_bytes=48 * (1 << 20),
        ),
    )(xb, wk, b[:, :co])
    return out.reshape(N, H // 2, W // 2, co)


def _ffc_block(x, w, b, tb=16, ck=2):
    N, H, W, dim = x.shape
    nb = H // tb
    xlp = jnp.pad(x[..., :64], ((0, 0), (1, 1), (1, 1), (0, 0)))
    xlb = _bands(xlp, tb, tb + 2, nb)
    xrb = x.reshape(N, nb, tb, W, dim)
    wl = w[:576, :256].reshape(9, 64, 256)
    wg = (w[576:768, 256:].astype(jnp.float32)
          + w[768:960, 256:].astype(jnp.float32)).astype(jnp.bfloat16)
    out = pl.pallas_call(
        functools.partial(_ffc_fused_kernel, tb=tb, ck=ck, wi=W),
        out_shape=jax.ShapeDtypeStruct((N, nb, tb, W, dim), jnp.bfloat16),
        grid=(N, nb),
        in_specs=[
            pl.BlockSpec((1, 1, tb + 2, W + 2, 64), lambda n, i: (n, i, 0, 0, 0)),
            pl.BlockSpec((1, 1, tb, W, dim), lambda n, i: (n, i, 0, 0, 0)),
            pl.BlockSpec((9, 64, 256), lambda n, i: (0, 0, 0)),
            pl.BlockSpec((192, 256), lambda n, i: (0, 0)),
            pl.BlockSpec((1, 512), lambda n, i: (0, 0)),
        ],
        out_specs=pl.BlockSpec((1, 1, tb, W, dim), lambda n, i: (n, i, 0, 0, 0)),
        compiler_params=pltpu.CompilerParams(
            dimension_semantics=("parallel", "parallel"),
            vmem_limit_bytes=48 * (1 << 20),
        ),
    )(xlb, xrb, wl, wg, b)
    return out.reshape(N, H, W, dim)


def _conv_up_fused(x, w, b, co, tb, ck):
    N, H, W, ci = x.shape
    nb = H // tb
    xp = jnp.pad(x, ((0, 0), (0, 1), (0, 1), (0, 0)))
    xb = _bands(xp, tb, tb + 1, nb)
    w4 = w[: 4 * ci]
    out = pl.pallas_call(
        functools.partial(_up_kernel, ci=ci, co=co, wo=W, tb=tb, ck=ck),
        out_shape=jax.ShapeDtypeStruct((N, nb, tb, 2, W, 2 * co), jnp.bfloat16),
        grid=(N, nb),
        in_specs=[
            pl.BlockSpec((1, 1, tb + 1, W + 1, ci), lambda n, i: (n, i, 0, 0, 0)),
            pl.BlockSpec((4 * ci, 4 * co), lambda n, i: (0, 0)),
            pl.BlockSpec((1, 4 * co), lambda n, i: (0, 0)),
        ],
        out_specs=pl.BlockSpec((1, 1, tb, 2, W, 2 * co),
                               lambda n, i: (n, i, 0, 0, 0, 0)),
        compiler_params=pltpu.CompilerParams(
            dimension_semantics=("parallel", "parallel"),
            vmem_limit_bytes=48 * (1 << 20),
        ),
    )(xb, w4, b[:, : 4 * co])
    return out.reshape(N, 2 * H, 2 * W, co)


def _im2col(x, k, stride):
    N, H, W, C = x.shape
    Ho = (H - k) // stride + 1
    Wo = (W - k) // stride + 1
    cols = [
        x[:, i : i + stride * (Ho - 1) + 1 : stride,
          j : j + stride * (Wo - 1) + 1 : stride, :]
        for i in range(k)
        for j in range(k)
    ]
    patches = jnp.stack(cols, axis=3).reshape(N * Ho * Wo, k * k * C)
    return patches, Ho, Wo


def _conv(x, w, b, cout, k, stride, pad, act, tm, out_dtype=jnp.bfloat16):
    if pad:
        x = jnp.pad(x, ((0, 0), (pad, pad), (pad, pad), (0, 0)))
    cols, Ho, Wo = _im2col(x, k, stride)
    out = _mm(cols, w, b, act, tm, out_dtype, n_out=cout)
    return out.reshape(x.shape[0], Ho, Wo, cout)


_GH, _GW = 4, 8
_HB = 8


def _head_weight(dec2_w, dec2_b):
    w4 = dec2_w[:3136, :3].astype(jnp.float32).reshape(7, 7, 64, 3)
    wg = jnp.zeros((_GH + 6, _GW + 6, 64, _GH, _GW, 3), jnp.float32)
    for dh in range(_GH):
        for dw in range(_GW):
            wg = wg.at[dh : dh + 7, dw : dw + 7, :, dh, dw, :].set(w4)
    wg = wg.reshape(_GH + 6, (_GW + 6) * 64, _GH * _GW * 3)
    wg = jnp.pad(wg, ((0, 0), (0, 0), (0, 128 - _GH * _GW * 3)))
    bias = jnp.tile(dec2_b[0, :3], _GH * _GW)
    bias = jnp.pad(bias, (0, 128 - _GH * _GW * 3)).reshape(1, -1)
    return wg.astype(jnp.bfloat16), bias.astype(jnp.float32)


def _head_kernel(x_ref, w_ref, b_ref, o_ref):
    xb = x_ref[0, 0].reshape(10, 4, 65, 512)
    acc = None
    for i2 in range(10):
        q, r = divmod(i2, 4)
        slab = xb[q : q + _HB, r]
        win = jnp.concatenate([slab[:, 0:64, :], slab[:, 1:65, :]], axis=2)
        pw = win[:, :, :896].reshape(_HB * 64, 896)
        d = jnp.dot(pw, w_ref[i2], preferred_element_type=jnp.float32)
        acc = d if acc is None else acc + d
    y = jax.nn.sigmoid(acc + b_ref[...])
    o_ref[0, 0] = y.reshape(_HB, 64, 128)


def _head_conv(x, wg, bg):
    N, H, W, _ = x.shape
    xp = jnp.pad(x, ((0, 0), (3, 3), (3, 3), (0, 0)), mode="reflect")
    xp = jnp.pad(xp, ((0, 0), (0, 2), (0, 2), (0, 0)))
    xr = xp.reshape(N, H + 8, (W + 8) // 8, 512)
    nb = H // (_GH * _HB)
    bands = jnp.stack([xr[:, 32 * b : 32 * b + 40] for b in range(nb)], axis=1)
    y = pl.pallas_call(
        _head_kernel,
        out_shape=jax.ShapeDtypeStruct((N, nb, _HB, 64, 128), jnp.float32),
        grid=(N, nb),
        in_specs=[
            pl.BlockSpec((1, 1, 40, 65, 512), lambda n, b: (n, b, 0, 0, 0)),
            pl.BlockSpec((10, 896, 128), lambda n, b: (0, 0, 0)),
            pl.BlockSpec((1, 128), lambda n, b: (0, 0)),
        ],
        out_specs=pl.BlockSpec((1, 1, _HB, 64, 128), lambda n, b: (n, b, 0, 0, 0)),
        compiler_params=pltpu.CompilerParams(
            dimension_semantics=("parallel", "parallel"),
            vmem_limit_bytes=48 * (1 << 20),
        ),
    )(bands, wg, bg)
    y = y[..., :96].reshape(N, nb, _HB, 64, _GH, _GW, 3)
    y = y.transpose(0, 1, 2, 4, 3, 5, 6)
    return y.reshape(N, H, W, 3)


def kernel(image, mask, enc0_w, enc0_b, enc1_w, enc1_b, enc2_w, enc2_b,
           mid0_w, mid0_b, mid1_w, mid1_b, dec0_w, dec0_b, dec1_w, dec1_b,
           dec2_w, dec2_b):
    ngf = 64
    img = jnp.transpose(image, (0, 2, 3, 1))
    msk = jnp.transpose(mask, (0, 2, 3, 1))
    keep = 1.0 - msk
    x = jnp.concatenate([img * keep, msk], axis=-1).astype(jnp.bfloat16)

    x = jnp.pad(x, ((0, 0), (3, 3), (3, 3), (0, 0)), mode="reflect")
    x = _enc0_conv(x, enc0_w, enc0_b)
    x = _conv_s2(x, enc1_w, enc1_b, ngf * 2, tb=8, ck=2)
    x = _conv_s2(x, enc2_w, enc2_b, ngf * 4, tb=8, ck=4)

    x = _ffc_block(x, mid0_w, mid0_b)
    x = _ffc_block(x, mid1_w, mid1_b)

    x = _conv_up_fused(x, dec0_w, dec0_b, ngf * 2, tb=16, ck=2)
    x = _conv_up_fused(x, dec1_w, dec1_b, ngf, tb=16, ck=2)
    wg, bg = _head_weight(dec2_w, dec2_b)
    y = _head_conv(x, wg, bg)

    out = img * keep + y * msk
    return {"inpainted": jnp.transpose(out, (0, 3, 1, 2))}


# speedup vs baseline: 1.1774x; 1.1774x over previous
import functools

import jax
import jax.numpy as jnp
from jax.experimental import pallas as pl
from jax.experimental.pallas import tpu as pltpu


def _ru(x, m):
    return (x + m - 1) // m * m


def _mm_act_kernel(x_ref, w_ref, b_ref, o_ref, *, act):
    y = jnp.dot(x_ref[...], w_ref[...], preferred_element_type=jnp.float32)
    y = y + b_ref[...]
    if act == "relu":
        y = jnp.maximum(y, 0.0)
    elif act == "sigmoid":
        y = jax.nn.sigmoid(y)
    o_ref[...] = y.astype(o_ref.dtype)


def _s2_kernel(x_ref, w_ref, b_ref, o_ref, *, ci, co, wo, tb, ck):
    xx = x_ref[0, 0].reshape(tb + 1, 2, wo + 1, 2, ci)
    for c0 in range(0, tb, ck):
        acc = None
        for i in range(3):
            qi, si = divmod(i, 2)
            for j in range(3):
                kj, pj = divmod(j, 2)
                slab = xx[c0 + qi : c0 + qi + ck, si, kj : kj + wo, pj, :]
                d = jnp.dot(slab.reshape(ck * wo, ci), w_ref[3 * i + j],
                            preferred_element_type=jnp.float32)
                acc = d if acc is None else acc + d
        y = jnp.maximum(acc + b_ref[...], 0.0)
        o_ref[0, 0, c0 : c0 + ck] = y.reshape(ck, wo, co).astype(o_ref.dtype)


def _ffc_fused_kernel(xl_ref, x_ref, wl_ref, wg_ref, b_ref, o_ref, *, tb, ck, wi):
    for c0 in range(0, tb, ck):
        xc = x_ref[0, 0, c0 : c0 + ck].reshape(ck * wi, 256)
        accg = jnp.dot(xc[:, 64:], wg_ref[...], preferred_element_type=jnp.float32)
        y1 = jnp.maximum(accg + b_ref[:, 256:], 0.0)
        accl = None
        for i in range(3):
            for j in range(3):
                slab = xl_ref[0, 0, c0 + i : c0 + i + ck, j : j + wi, :]
                d = jnp.dot(slab.reshape(ck * wi, 64), wl_ref[3 * i + j],
                            preferred_element_type=jnp.float32)
                accl = d if accl is None else accl + d
        y0 = jnp.maximum(accl + b_ref[:, :256], 0.0)
        o = y0 + y1 + xc.astype(jnp.float32)
        o_ref[0, 0, c0 : c0 + ck] = o.reshape(ck, wi, 256).astype(o_ref.dtype)


def _up_kernel(x_ref, w_ref, b_ref, o_ref, *, ci, co, wo, tb, ck):
    for c0 in range(0, tb, ck):
        acc = None
        for t, (i, j) in enumerate(((0, 0), (0, 1), (1, 0), (1, 1))):
            slab = x_ref[0, 0, c0 + i : c0 + i + ck, j : j + wo, :]
            d = jnp.dot(slab.reshape(ck * wo, ci), w_ref[t * ci : (t + 1) * ci],
                        preferred_element_type=jnp.float32)
            acc = d if acc is None else acc + d
        y = jnp.maximum(acc + b_ref[...], 0.0).astype(o_ref.dtype)
        y = y.reshape(ck, wo, 4 * co)
        for a in (0, 1):
            o_ref[0, 0, c0 : c0 + ck, a] = y[:, :, a * 2 * co : (a + 1) * 2 * co]


def _vmem_budget(*arrs):
    need = sum(2 * a.size * a.dtype.itemsize for a in arrs)
    return int(min(60 * (1 << 20), max(32 * (1 << 20), need)))


def _mm(x, w, b, act, tm, out_dtype=jnp.bfloat16, n_out=None):
    M, K = x.shape
    Kp, Np = w.shape
    Mp = _ru(M, tm)
    xb = x.astype(jnp.bfloat16)
    if (Mp, Kp) != (M, K):
        xb = jnp.pad(xb, ((0, Mp - M), (0, Kp - K)))
    out = pl.pallas_call(
        functools.partial(_mm_act_kernel, act=act),
        out_shape=jax.ShapeDtypeStruct((Mp, Np), out_dtype),
        grid=(Mp // tm,),
        in_specs=[
            pl.BlockSpec((tm, Kp), lambda i: (i, 0)),
            pl.BlockSpec((Kp, Np), lambda i: (0, 0)),
            pl.BlockSpec((1, Np), lambda i: (0, 0)),
        ],
        out_specs=pl.BlockSpec((tm, Np), lambda i: (i, 0)),
        compiler_params=pltpu.CompilerParams(
            dimension_semantics=("parallel",),
            vmem_limit_bytes=_vmem_budget(
                jax.ShapeDtypeStruct((tm, Kp), jnp.bfloat16),
                jax.ShapeDtypeStruct((tm, Np), out_dtype),
                jax.ShapeDtypeStruct((Kp, Np), jnp.bfloat16),
            ),
        ),
    )(xb, w, b)
    if n_out is None and Mp == M:
        return out
    return out[:M, : (Np if n_out is None else n_out)]


def _enc0_kernel(x_ref, w_ref, b_ref, o_ref, *, tb):
    wins = []
    for i in range(7):
        slab = x_ref[0, 0, i : i + tb]
        wins.append(jnp.concatenate([slab[:, 0:16, :], slab[:, 1:17, :]], axis=2))
    p = jnp.concatenate(wins, axis=2).reshape(tb * 16, 7 * 256)
    y = jnp.dot(p, w_ref[...], preferred_element_type=jnp.float32)
    y = jnp.maximum(y + b_ref[...], 0.0)
    o_ref[0, 0] = y.reshape(tb, 16, 2048).astype(o_ref.dtype)


def _enc0_conv(x, w, b, tb=8):
    N = x.shape[0]
    xp = jnp.pad(x, ((0, 0), (0, 0), (0, 26), (0, 0))).reshape(N, 518, 17, 128)
    nb = 512 // tb
    xb = _bands(xp, tb, tb + 6, nb)
    w7 = w[:196, :64].astype(jnp.float32).reshape(7, 7, 4, 64)
    wg = jnp.zeros((7, 64, 4, 32, 64), jnp.float32)
    for t in range(32):
        wg = wg.at[:, t : t + 7, :, t, :].set(w7)
    wg = wg.reshape(1792, 2048).astype(jnp.bfloat16)
    bg = jnp.tile(b[0, :64], 32).reshape(1, 2048)
    out = pl.pallas_call(
        functools.partial(_enc0_kernel, tb=tb),
        out_shape=jax.ShapeDtypeStruct((N, nb, tb, 16, 2048), jnp.bfloat16),
        grid=(N, nb),
        in_specs=[
            pl.BlockSpec((1, 1, tb + 6, 17, 128), lambda n, i: (n, i, 0, 0, 0)),
            pl.BlockSpec((1792, 2048), lambda n, i: (0, 0)),
            pl.BlockSpec((1, 2048), lambda n, i: (0, 0)),
        ],
        out_specs=pl.BlockSpec((1, 1, tb, 16, 2048), lambda n, i: (n, i, 0, 0, 0)),
        compiler_params=pltpu.CompilerParams(
            dimension_semantics=("parallel", "parallel"),
            vmem_limit_bytes=48 * (1 << 20),
        ),
    )(xb, wg, bg)
    return out.reshape(N, 512, 512, 64)


def _bands(x, row0_stride, rows_in, nb):
    return jnp.stack([x[:, row0_stride * b : row0_stride * b + rows_in]
                      for b in range(nb)], axis=1)


def _conv_s2(x, w, b, co, tb, ck):
    N, H, W, ci = x.shape
    wo = W // 2
    nb = (H // 2) // tb
    xp = jnp.pad(x, ((0, 0), (1, 1), (1, 1), (0, 0)))
    xb = _bands(xp, 2 * tb, 2 * tb + 2, nb)
    w9 = w[: 9 * ci].reshape(9, ci, co)
    out = pl.pallas_call(
        functools.partial(_s2_kernel, ci=ci, co=co, wo=wo, tb=tb, ck=ck),
        out_shape=jax.ShapeDtypeStruct((N, nb, tb, wo, co), jnp.bfloat16),
        grid=(N, nb),
        in_specs=[
            pl.BlockSpec((1, 1, 2 * tb + 2, 2 * wo + 2, ci),
                         lambda n, i: (n, i, 0, 0, 0)),
            pl.BlockSpec((9, ci, co), lambda n, i: (0, 0, 0)),
            pl.BlockSpec((1, co), lambda n, i: (0, 0)),
        ],
        out_specs=pl.BlockSpec((1, 1, tb, wo, co), lambda n, i: (n, i, 0, 0, 0)),
        compiler_params=pltpu.CompilerParams(
            dimension_semantics=("parallel", "parallel"),
            vmem_limit_bytes=48 * (1 << 20),
        ),
    )(xb, w9, b[:, :co])
    return out.reshape(N, H // 2, W // 2, co)


def _ffc_block(x, w, b, tb=16, ck=2):
    N, H, W, dim = x.shape
    nb = H // tb
    xlp = jnp.pad(x[..., :64], ((0, 0), (1, 1), (1, 1), (0, 0)))
    xlb = _bands(xlp, tb, tb + 2, nb)
    xrb = x.reshape(N, nb, tb, W, dim)
    wl = w[:576, :256].reshape(9, 64, 256)
    wg = (w[576:768, 256:].astype(jnp.float32)
          + w[768:960, 256:].astype(jnp.float32)).astype(jnp.bfloat16)
    out = pl.pallas_call(
        functools.partial(_ffc_fused_kernel, tb=tb, ck=ck, wi=W),
        out_shape=jax.ShapeDtypeStruct((N, nb, tb, W, dim), jnp.bfloat16),
        grid=(N, nb),
        in_specs=[
            pl.BlockSpec((1, 1, tb + 2, W + 2, 64), lambda n, i: (n, i, 0, 0, 0)),
            pl.BlockSpec((1, 1, tb, W, dim), lambda n, i: (n, i, 0, 0, 0)),
            pl.BlockSpec((9, 64, 256), lambda n, i: (0, 0, 0)),
            pl.BlockSpec((192, 256), lambda n, i: (0, 0)),
            pl.BlockSpec((1, 512), lambda n, i: (0, 0)),
        ],
        out_specs=pl.BlockSpec((1, 1, tb, W, dim), lambda n, i: (n, i, 0, 0, 0)),
        compiler_params=pltpu.CompilerParams(
            dimension_semantics=("parallel", "parallel"),
            vmem_limit_bytes=48 * (1 << 20),
        ),
    )(xlb, xrb, wl, wg, b)
    return out.reshape(N, H, W, dim)


def _conv_up_fused(x, w, b, co, tb, ck):
    N, H, W, ci = x.shape
    nb = H // tb
    xp = jnp.pad(x, ((0, 0), (0, 1), (0, 1), (0, 0)))
    xb = _bands(xp, tb, tb + 1, nb)
    w4 = w[: 4 * ci]
    out = pl.pallas_call(
        functools.partial(_up_kernel, ci=ci, co=co, wo=W, tb=tb, ck=ck),
        out_shape=jax.ShapeDtypeStruct((N, nb, tb, 2, W, 2 * co), jnp.bfloat16),
        grid=(N, nb),
        in_specs=[
            pl.BlockSpec((1, 1, tb + 1, W + 1, ci), lambda n, i: (n, i, 0, 0, 0)),
            pl.BlockSpec((4 * ci, 4 * co), lambda n, i: (0, 0)),
            pl.BlockSpec((1, 4 * co), lambda n, i: (0, 0)),
        ],
        out_specs=pl.BlockSpec((1, 1, tb, 2, W, 2 * co),
                               lambda n, i: (n, i, 0, 0, 0, 0)),
        compiler_params=pltpu.CompilerParams(
            dimension_semantics=("parallel", "parallel"),
            vmem_limit_bytes=48 * (1 << 20),
        ),
    )(xb, w4, b[:, : 4 * co])
    return out.reshape(N, 2 * H, 2 * W, co)


def _im2col(x, k, stride):
    N, H, W, C = x.shape
    Ho = (H - k) // stride + 1
    Wo = (W - k) // stride + 1
    cols = [
        x[:, i : i + stride * (Ho - 1) + 1 : stride,
          j : j + stride * (Wo - 1) + 1 : stride, :]
        for i in range(k)
        for j in range(k)
    ]
    patches = jnp.stack(cols, axis=3).reshape(N * Ho * Wo, k * k * C)
    return patches, Ho, Wo


def _conv(x, w, b, cout, k, stride, pad, act, tm, out_dtype=jnp.bfloat16):
    if pad:
        x = jnp.pad(x, ((0, 0), (pad, pad), (pad, pad), (0, 0)))
    cols, Ho, Wo = _im2col(x, k, stride)
    out = _mm(cols, w, b, act, tm, out_dtype, n_out=cout)
    return out.reshape(x.shape[0], Ho, Wo, cout)


_GH, _GW = 4, 8
_HB = 8


def _head_weight(dec2_w, dec2_b):
    w4 = dec2_w[:3136, :3].astype(jnp.float32).reshape(7, 7, 64, 3)
    wg = jnp.zeros((_GH + 6, _GW + 6, 64, _GH, _GW, 3), jnp.float32)
    for dh in range(_GH):
        for dw in range(_GW):
            wg = wg.at[dh : dh + 7, dw : dw + 7, :, dh, dw, :].set(w4)
    wg = wg.reshape(_GH + 6, (_GW + 6) * 64, _GH * _GW * 3)
    wg = jnp.pad(wg, ((0, 0), (0, 0), (0, 128 - _GH * _GW * 3)))
    bias = jnp.tile(dec2_b[0, :3], _GH * _GW)
    bias = jnp.pad(bias, (0, 128 - _GH * _GW * 3)).reshape(1, -1)
    return wg.astype(jnp.bfloat16), bias.astype(jnp.float32)


def _head_kernel(x_ref, w_ref, b_ref, o_ref):
    xb = x_ref[0, 0].reshape(10, 4, 65, 512)
    acc = None
    for i2 in range(10):
        q, r = divmod(i2, 4)
        slab = xb[q : q + _HB, r]
        win = jnp.concatenate([slab[:, 0:64, :], slab[:, 1:65, :]], axis=2)
        pw = win[:, :, :896].reshape(_HB * 64, 896)
        d = jnp.dot(pw, w_ref[i2], preferred_element_type=jnp.float32)
        acc = d if acc is None else acc + d
    y = jax.nn.sigmoid(acc + b_ref[...])
    o_ref[0, 0] = y.reshape(_HB, 64, 128)


def _head_conv(x, wg, bg):
    N, H, W, _ = x.shape
    xp = jnp.pad(x, ((0, 0), (3, 3), (3, 3), (0, 0)), mode="reflect")
    xp = jnp.pad(xp, ((0, 0), (0, 2), (0, 2), (0, 0)))
    xr = xp.reshape(N, H + 8, (W + 8) // 8, 512)
    nb = H // (_GH * _HB)
    bands = jnp.stack([xr[:, 32 * b : 32 * b + 40] for b in range(nb)], axis=1)
    y = pl.pallas_call(
        _head_kernel,
        out_shape=jax.ShapeDtypeStruct((N, nb, _HB, 64, 128), jnp.float32),
        grid=(N, nb),
        in_specs=[
            pl.BlockSpec((1, 1, 40, 65, 512), lambda n, b: (n, b, 0, 0, 0)),
            pl.BlockSpec((10, 896, 128), lambda n, b: (0, 0, 0)),
            pl.BlockSpec((1, 128), lambda n, b: (0, 0)),
        ],
        out_specs=pl.BlockSpec((1, 1, _HB, 64, 128), lambda n, b: (n, b, 0, 0, 0)),
        compiler_params=pltpu.CompilerParams(
            dimension_semantics=("parallel", "parallel"),
            vmem_limit_bytes=48 * (1 << 20),
        ),
    )(bands, wg, bg)
    y = y[..., :96].reshape(N, nb, _HB, 64, _GH, _GW, 3)
    y = y.transpose(0, 1, 2, 4, 3, 5, 6)
    return y.reshape(N, H, W, 3)


def kernel(image, mask, enc0_w, enc0_b, enc1_w, enc1_b, enc2_w, enc2_b,
           mid0_w, mid0_b, mid1_w, mid1_b, dec0_w, dec0_b, dec1_w, dec1_b,
           dec2_w, dec2_b):
    ngf = 64
    img = jnp.transpose(image, (0, 2, 3, 1))
    msk = jnp.transpose(mask, (0, 2, 3, 1))
    keep = 1.0 - msk
    x = jnp.concatenate([img * keep, msk], axis=-1).astype(jnp.bfloat16)

    x = jnp.pad(x, ((0, 0), (3, 3), (3, 3), (0, 0)), mode="reflect")
    x = _enc0_conv(x, enc0_w, enc0_b)
    x = _conv_s2(x, enc1_w, enc1_b, ngf * 2, tb=8, ck=2)
    x = _conv_s2(x, enc2_w, enc2_b, ngf * 4, tb=8, ck=4)

    x = _ffc_block(x, mid0_w, mid0_b)
    x = _ffc_block(x, mid1_w, mid1_b)

    x = _conv_up_fused(x, dec0_w, dec0_b, ngf * 2, tb=16, ck=2)
    x = _conv_up_fused(x, dec1_w, dec1_b, ngf, tb=16, ck=2)
    wg, bg = _head_weight(dec2_w, dec2_b)
    y = _head_conv(x, wg, bg)

    out = img * keep + y * msk
    return {"inpainted": jnp.transpose(out, (0, 3, 1, 2))}


# speedup vs baseline: 1.2275x; 1.0425x over previous
import functools

import jax
import jax.numpy as jnp
from jax.experimental import pallas as pl
from jax.experimental.pallas import tpu as pltpu


def _ru(x, m):
    return (x + m - 1) // m * m


def _mm_act_kernel(x_ref, w_ref, b_ref, o_ref, *, act):
    y = jnp.dot(x_ref[...], w_ref[...], preferred_element_type=jnp.float32)
    y = y + b_ref[...]
    if act == "relu":
        y = jnp.maximum(y, 0.0)
    elif act == "sigmoid":
        y = jax.nn.sigmoid(y)
    o_ref[...] = y.astype(o_ref.dtype)


def _s2_kernel(x_ref, w_ref, b_ref, o_ref, *, ci, co, wo, tb, ck):
    xx = x_ref[0, 0].reshape(tb + 1, 2, wo + 1, 2, ci)
    for c0 in range(0, tb, ck):
        acc = None
        for i in range(3):
            qi, si = divmod(i, 2)
            for j in range(3):
                kj, pj = divmod(j, 2)
                slab = xx[c0 + qi : c0 + qi + ck, si, kj : kj + wo, pj, :]
                d = jnp.dot(slab.reshape(ck * wo, ci), w_ref[3 * i + j],
                            preferred_element_type=jnp.float32)
                acc = d if acc is None else acc + d
        y = jnp.maximum(acc + b_ref[...], 0.0)
        o_ref[0, 0, c0 : c0 + ck] = y.reshape(ck, wo, co).astype(o_ref.dtype)


def _ffc_fused_kernel(xl_ref, x_ref, wl_ref, wg_ref, b_ref, o_ref, *, tb, ck, wi):
    for c0 in range(0, tb, ck):
        xc = x_ref[0, 0, c0 : c0 + ck].reshape(ck * wi, 256)
        accg = jnp.dot(xc[:, 64:], wg_ref[...], preferred_element_type=jnp.float32)
        y1 = jnp.maximum(accg + b_ref[:, 256:], 0.0)
        accl = None
        for i in range(3):
            for j in range(3):
                slab = xl_ref[0, 0, c0 + i : c0 + i + ck, j : j + wi, :]
                d = jnp.dot(slab.reshape(ck * wi, 64), wl_ref[3 * i + j],
                            preferred_element_type=jnp.float32)
                accl = d if accl is None else accl + d
        y0 = jnp.maximum(accl + b_ref[:, :256], 0.0)
        o = y0 + y1 + xc.astype(jnp.float32)
        o_ref[0, 0, c0 : c0 + ck] = o.reshape(ck, wi, 256).astype(o_ref.dtype)


def _up_kernel(x_ref, w_ref, b_ref, o_ref, *, ci, co, wo, tb, ck):
    for c0 in range(0, tb, ck):
        acc = None
        for t, (i, j) in enumerate(((0, 0), (0, 1), (1, 0), (1, 1))):
            slab = x_ref[0, 0, c0 + i : c0 + i + ck, j : j + wo, :]
            d = jnp.dot(slab.reshape(ck * wo, ci), w_ref[t * ci : (t + 1) * ci],
                        preferred_element_type=jnp.float32)
            acc = d if acc is None else acc + d
        y = jnp.maximum(acc + b_ref[...], 0.0).astype(o_ref.dtype)
        y = y.reshape(ck, wo, 4 * co)
        for a in (0, 1):
            o_ref[0, 0, c0 : c0 + ck, a] = y[:, :, a * 2 * co : (a + 1) * 2 * co]


def _vmem_budget(*arrs):
    need = sum(2 * a.size * a.dtype.itemsize for a in arrs)
    return int(min(60 * (1 << 20), max(32 * (1 << 20), need)))


def _mm(x, w, b, act, tm, out_dtype=jnp.bfloat16, n_out=None):
    M, K = x.shape
    Kp, Np = w.shape
    Mp = _ru(M, tm)
    xb = x.astype(jnp.bfloat16)
    if (Mp, Kp) != (M, K):
        xb = jnp.pad(xb, ((0, Mp - M), (0, Kp - K)))
    out = pl.pallas_call(
        functools.partial(_mm_act_kernel, act=act),
        out_shape=jax.ShapeDtypeStruct((Mp, Np), out_dtype),
        grid=(Mp // tm,),
        in_specs=[
            pl.BlockSpec((tm, Kp), lambda i: (i, 0)),
            pl.BlockSpec((Kp, Np), lambda i: (0, 0)),
            pl.BlockSpec((1, Np), lambda i: (0, 0)),
        ],
        out_specs=pl.BlockSpec((tm, Np), lambda i: (i, 0)),
        compiler_params=pltpu.CompilerParams(
            dimension_semantics=("parallel",),
            vmem_limit_bytes=_vmem_budget(
                jax.ShapeDtypeStruct((tm, Kp), jnp.bfloat16),
                jax.ShapeDtypeStruct((tm, Np), out_dtype),
                jax.ShapeDtypeStruct((Kp, Np), jnp.bfloat16),
            ),
        ),
    )(xb, w, b)
    if n_out is None and Mp == M:
        return out
    return out[:M, : (Np if n_out is None else n_out)]


def _enc0_kernel(x_ref, w_ref, b_ref, o_ref, *, tb):
    wins = []
    for i in range(7):
        slab = x_ref[0, 0, i : i + tb]
        wins.append(jnp.concatenate([slab[:, 0:16, :], slab[:, 1:17, :]], axis=2))
    p = jnp.concatenate(wins, axis=2).reshape(tb * 16, 7 * 256)
    y = jnp.dot(p, w_ref[...], preferred_element_type=jnp.float32)
    y = jnp.maximum(y + b_ref[...], 0.0)
    o_ref[0, 0] = y.reshape(tb, 16, 2048).astype(o_ref.dtype)


def _enc0_conv(x, w, b, tb=16):
    N = x.shape[0]
    xp = jnp.pad(x, ((0, 0), (0, 0), (0, 26), (0, 0))).reshape(N, 518, 17, 128)
    nb = 512 // tb
    xb = _bands(xp, tb, tb + 6, nb)
    w7 = w[:196, :64].astype(jnp.float32).reshape(7, 7, 4, 64)
    wg = jnp.zeros((7, 64, 4, 32, 64), jnp.float32)
    for t in range(32):
        wg = wg.at[:, t : t + 7, :, t, :].set(w7)
    wg = wg.reshape(1792, 2048).astype(jnp.bfloat16)
    bg = jnp.tile(b[0, :64], 32).reshape(1, 2048)
    out = pl.pallas_call(
        functools.partial(_enc0_kernel, tb=tb),
        out_shape=jax.ShapeDtypeStruct((N, nb, tb, 16, 2048), jnp.bfloat16),
        grid=(N, nb),
        in_specs=[
            pl.BlockSpec((1, 1, tb + 6, 17, 128), lambda n, i: (n, i, 0, 0, 0)),
            pl.BlockSpec((1792, 2048), lambda n, i: (0, 0)),
            pl.BlockSpec((1, 2048), lambda n, i: (0, 0)),
        ],
        out_specs=pl.BlockSpec((1, 1, tb, 16, 2048), lambda n, i: (n, i, 0, 0, 0)),
        compiler_params=pltpu.CompilerParams(
            dimension_semantics=("parallel", "parallel"),
            vmem_limit_bytes=48 * (1 << 20),
        ),
    )(xb, wg, bg)
    return out.reshape(N, 512, 512, 64)


def _bands(x, row0_stride, rows_in, nb):
    return jnp.stack([x[:, row0_stride * b : row0_stride * b + rows_in]
                      for b in range(nb)], axis=1)


def _conv_s2(x, w, b, co, tb, ck):
    N, H, W, ci = x.shape
    wo = W // 2
    nb = (H // 2) // tb
    xp = jnp.pad(x, ((0, 0), (1, 1), (1, 1), (0, 0)))
    xb = _bands(xp, 2 * tb, 2 * tb + 2, nb)
    w9 = w[: 9 * ci].reshape(9, ci, co)
    out = pl.pallas_call(
        functools.partial(_s2_kernel, ci=ci, co=co, wo=wo, tb=tb, ck=ck),
        out_shape=jax.ShapeDtypeStruct((N, nb, tb, wo, co), jnp.bfloat16),
        grid=(N, nb),
        in_specs=[
            pl.BlockSpec((1, 1, 2 * tb + 2, 2 * wo + 2, ci),
                         lambda n, i: (n, i, 0, 0, 0)),
            pl.BlockSpec((9, ci, co), lambda n, i: (0, 0, 0)),
            pl.BlockSpec((1, co), lambda n, i: (0, 0)),
        ],
        out_specs=pl.BlockSpec((1, 1, tb, wo, co), lambda n, i: (n, i, 0, 0, 0)),
        compiler_params=pltpu.CompilerParams(
            dimension_semantics=("parallel", "parallel"),
            vmem_limit_bytes=48 * (1 << 20),
        ),
    )(xb, w9, b[:, :co])
    return out.reshape(N, H // 2, W // 2, co)


def _ffc_block(x, w, b, tb=16, ck=4):
    N, H, W, dim = x.shape
    nb = H // tb
    xlp = jnp.pad(x[..., :64], ((0, 0), (1, 1), (1, 1), (0, 0)))
    xlb = _bands(xlp, tb, tb + 2, nb)
    xrb = x.reshape(N, nb, tb, W, dim)
    wl = w[:576, :256].reshape(9, 64, 256)
    wg = (w[576:768, 256:].astype(jnp.float32)
          + w[768:960, 256:].astype(jnp.float32)).astype(jnp.bfloat16)
    out = pl.pallas_call(
        functools.partial(_ffc_fused_kernel, tb=tb, ck=ck, wi=W),
        out_shape=jax.ShapeDtypeStruct((N, nb, tb, W, dim), jnp.bfloat16),
        grid=(N, nb),
        in_specs=[
            pl.BlockSpec((1, 1, tb + 2, W + 2, 64), lambda n, i: (n, i, 0, 0, 0)),
            pl.BlockSpec((1, 1, tb, W, dim), lambda n, i: (n, i, 0, 0, 0)),
            pl.BlockSpec((9, 64, 256), lambda n, i: (0, 0, 0)),
            pl.BlockSpec((192, 256), lambda n, i: (0, 0)),
            pl.BlockSpec((1, 512), lambda n, i: (0, 0)),
        ],
        out_specs=pl.BlockSpec((1, 1, tb, W, dim), lambda n, i: (n, i, 0, 0, 0)),
        compiler_params=pltpu.CompilerParams(
            dimension_semantics=("parallel", "parallel"),
            vmem_limit_bytes=48 * (1 << 20),
        ),
    )(xlb, xrb, wl, wg, b)
    return out.reshape(N, H, W, dim)


def _conv_up_fused(x, w, b, co, tb, ck):
    N, H, W, ci = x.shape
    nb = H // tb
    xp = jnp.pad(x, ((0, 0), (0, 1), (0, 1), (0, 0)))
    xb = _bands(xp, tb, tb + 1, nb)
    w4 = w[: 4 * ci]
    out = pl.pallas_call(
        functools.partial(_up_kernel, ci=ci, co=co, wo=W, tb=tb, ck=ck),
        out_shape=jax.ShapeDtypeStruct((N, nb, tb, 2, W, 2 * co), jnp.bfloat16),
        grid=(N, nb),
        in_specs=[
            pl.BlockSpec((1, 1, tb + 1, W + 1, ci), lambda n, i: (n, i, 0, 0, 0)),
            pl.BlockSpec((4 * ci, 4 * co), lambda n, i: (0, 0)),
            pl.BlockSpec((1, 4 * co), lambda n, i: (0, 0)),
        ],
        out_specs=pl.BlockSpec((1, 1, tb, 2, W, 2 * co),
                               lambda n, i: (n, i, 0, 0, 0, 0)),
        compiler_params=pltpu.CompilerParams(
            dimension_semantics=("parallel", "parallel"),
            vmem_limit_bytes=48 * (1 << 20),
        ),
    )(xb, w4, b[:, : 4 * co])
    return out.reshape(N, 2 * H, 2 * W, co)


def _im2col(x, k, stride):
    N, H, W, C = x.shape
    Ho = (H - k) // stride + 1
    Wo = (W - k) // stride + 1
    cols = [
        x[:, i : i + stride * (Ho - 1) + 1 : stride,
          j : j + stride * (Wo - 1) + 1 : stride, :]
        for i in range(k)
        for j in range(k)
    ]
    patches = jnp.stack(cols, axis=3).reshape(N * Ho * Wo, k * k * C)
    return patches, Ho, Wo


def _conv(x, w, b, cout, k, stride, pad, act, tm, out_dtype=jnp.bfloat16):
    if pad:
        x = jnp.pad(x, ((0, 0), (pad, pad), (pad, pad), (0, 0)))
    cols, Ho, Wo = _im2col(x, k, stride)
    out = _mm(cols, w, b, act, tm, out_dtype, n_out=cout)
    return out.reshape(x.shape[0], Ho, Wo, cout)


_GH, _GW = 4, 8
_HB = 8


def _head_weight(dec2_w, dec2_b):
    w4 = dec2_w[:3136, :3].astype(jnp.float32).reshape(7, 7, 64, 3)
    wg = jnp.zeros((_GH + 6, _GW + 6, 64, _GH, _GW, 3), jnp.float32)
    for dh in range(_GH):
        for dw in range(_GW):
            wg = wg.at[dh : dh + 7, dw : dw + 7, :, dh, dw, :].set(w4)
    wg = wg.reshape(_GH + 6, (_GW + 6) * 64, _GH * _GW * 3)
    wg = jnp.pad(wg, ((0, 0), (0, 0), (0, 128 - _GH * _GW * 3)))
    bias = jnp.tile(dec2_b[0, :3], _GH * _GW)
    bias = jnp.pad(bias, (0, 128 - _GH * _GW * 3)).reshape(1, -1)
    return wg.astype(jnp.bfloat16), bias.astype(jnp.float32)


def _head_kernel(x_ref, w_ref, b_ref, o_ref):
    xb = x_ref[0, 0].reshape(10, 4, 65, 512)
    acc = None
    for i2 in range(10):
        q, r = divmod(i2, 4)
        slab = xb[q : q + _HB, r]
        win = jnp.concatenate([slab[:, 0:64, :], slab[:, 1:65, :]], axis=2)
        pw = win[:, :, :896].reshape(_HB * 64, 896)
        d = jnp.dot(pw, w_ref[i2], preferred_element_type=jnp.float32)
        acc = d if acc is None else acc + d
    y = jax.nn.sigmoid(acc + b_ref[...])
    o_ref[0, 0] = y.reshape(_HB, 64, 128)


def _head_conv(x, wg, bg):
    N, H, W, _ = x.shape
    xp = jnp.pad(x, ((0, 0), (3, 3), (3, 3), (0, 0)), mode="reflect")
    xp = jnp.pad(xp, ((0, 0), (0, 2), (0, 2), (0, 0)))
    xr = xp.reshape(N, H + 8, (W + 8) // 8, 512)
    nb = H // (_GH * _HB)
    bands = jnp.stack([xr[:, 32 * b : 32 * b + 40] for b in range(nb)], axis=1)
    y = pl.pallas_call(
        _head_kernel,
        out_shape=jax.ShapeDtypeStruct((N, nb, _HB, 64, 128), jnp.float32),
        grid=(N, nb),
        in_specs=[
            pl.BlockSpec((1, 1, 40, 65, 512), lambda n, b: (n, b, 0, 0, 0)),
            pl.BlockSpec((10, 896, 128), lambda n, b: (0, 0, 0)),
            pl.BlockSpec((1, 128), lambda n, b: (0, 0)),
        ],
        out_specs=pl.BlockSpec((1, 1, _HB, 64, 128), lambda n, b: (n, b, 0, 0, 0)),
        compiler_params=pltpu.CompilerParams(
            dimension_semantics=("parallel", "parallel"),
            vmem_limit_bytes=48 * (1 << 20),
        ),
    )(bands, wg, bg)
    y = y[..., :96].reshape(N, nb, _HB, 64, _GH, _GW, 3)
    y = y.transpose(0, 1, 2, 4, 3, 5, 6)
    return y.reshape(N, H, W, 3)


def kernel(image, mask, enc0_w, enc0_b, enc1_w, enc1_b, enc2_w, enc2_b,
           mid0_w, mid0_b, mid1_w, mid1_b, dec0_w, dec0_b, dec1_w, dec1_b,
           dec2_w, dec2_b):
    ngf = 64
    img = jnp.transpose(image, (0, 2, 3, 1))
    msk = jnp.transpose(mask, (0, 2, 3, 1))
    keep = 1.0 - msk
    x = jnp.concatenate([img * keep, msk], axis=-1).astype(jnp.bfloat16)

    x = jnp.pad(x, ((0, 0), (3, 3), (3, 3), (0, 0)), mode="reflect")
    x = _enc0_conv(x, enc0_w, enc0_b)
    x = _conv_s2(x, enc1_w, enc1_b, ngf * 2, tb=8, ck=4)
    x = _conv_s2(x, enc2_w, enc2_b, ngf * 4, tb=8, ck=4)

    x = _ffc_block(x, mid0_w, mid0_b)
    x = _ffc_block(x, mid1_w, mid1_b)

    x = _conv_up_fused(x, dec0_w, dec0_b, ngf * 2, tb=16, ck=2)
    x = _conv_up_fused(x, dec1_w, dec1_b, ngf, tb=16, ck=2)
    wg, bg = _head_weight(dec2_w, dec2_b)
    y = _head_conv(x, wg, bg)

    out = img * keep + y * msk
    return {"inpainted": jnp.transpose(out, (0, 3, 1, 2))}


# speedup vs baseline: 1.2544x; 1.0219x over previous
import functools

import jax
import jax.numpy as jnp
from jax.experimental import pallas as pl
from jax.experimental.pallas import tpu as pltpu


def _ru(x, m):
    return (x + m - 1) // m * m


def _mm_act_kernel(x_ref, w_ref, b_ref, o_ref, *, act):
    y = jnp.dot(x_ref[...], w_ref[...], preferred_element_type=jnp.float32)
    y = y + b_ref[...]
    if act == "relu":
        y = jnp.maximum(y, 0.0)
    elif act == "sigmoid":
        y = jax.nn.sigmoid(y)
    o_ref[...] = y.astype(o_ref.dtype)


def _s2_kernel(x_ref, w_ref, b_ref, o_ref, *, ci, co, wo, tb, ck):
    xx = x_ref[0, 0].reshape(tb + 1, 2, wo + 1, 2, ci)
    for c0 in range(0, tb, ck):
        acc = None
        for i in range(3):
            qi, si = divmod(i, 2)
            for j in range(3):
                kj, pj = divmod(j, 2)
                slab = xx[c0 + qi : c0 + qi + ck, si, kj : kj + wo, pj, :]
                d = jnp.dot(slab.reshape(ck * wo, ci), w_ref[3 * i + j],
                            preferred_element_type=jnp.float32)
                acc = d if acc is None else acc + d
        y = jnp.maximum(acc + b_ref[...], 0.0)
        o_ref[0, 0, c0 : c0 + ck] = y.reshape(ck, wo, co).astype(o_ref.dtype)


def _ffc_fused_kernel(xl_ref, x_ref, wl_ref, wg_ref, b_ref, o_ref, *, tb, ck, wi):
    for c0 in range(0, tb, ck):
        xc = x_ref[0, 0, c0 : c0 + ck].reshape(ck * wi, 256)
        accg = jnp.dot(xc[:, 64:], wg_ref[...], preferred_element_type=jnp.float32)
        y1 = jnp.maximum(accg + b_ref[:, 256:], 0.0)
        accl = None
        for i in range(3):
            for j in range(3):
                slab = xl_ref[0, 0, c0 + i : c0 + i + ck, j : j + wi, :]
                d = jnp.dot(slab.reshape(ck * wi, 64), wl_ref[3 * i + j],
                            preferred_element_type=jnp.float32)
                accl = d if accl is None else accl + d
        y0 = jnp.maximum(accl + b_ref[:, :256], 0.0)
        o = y0 + y1 + xc.astype(jnp.float32)
        o_ref[0, 0, c0 : c0 + ck] = o.reshape(ck, wi, 256).astype(o_ref.dtype)


def _up_kernel(x_ref, w_ref, b_ref, o_ref, *, ci, co, wo, tb, ck):
    for c0 in range(0, tb, ck):
        acc = None
        for t, (i, j) in enumerate(((0, 0), (0, 1), (1, 0), (1, 1))):
            slab = x_ref[0, 0, c0 + i : c0 + i + ck, j : j + wo, :]
            d = jnp.dot(slab.reshape(ck * wo, ci), w_ref[t * ci : (t + 1) * ci],
                        preferred_element_type=jnp.float32)
            acc = d if acc is None else acc + d
        y = jnp.maximum(acc + b_ref[...], 0.0).astype(o_ref.dtype)
        y = y.reshape(ck, wo, 4 * co)
        for a in (0, 1):
            o_ref[0, 0, c0 : c0 + ck, a] = y[:, :, a * 2 * co : (a + 1) * 2 * co]


def _vmem_budget(*arrs):
    need = sum(2 * a.size * a.dtype.itemsize for a in arrs)
    return int(min(60 * (1 << 20), max(32 * (1 << 20), need)))


def _mm(x, w, b, act, tm, out_dtype=jnp.bfloat16, n_out=None):
    M, K = x.shape
    Kp, Np = w.shape
    Mp = _ru(M, tm)
    xb = x.astype(jnp.bfloat16)
    if (Mp, Kp) != (M, K):
        xb = jnp.pad(xb, ((0, Mp - M), (0, Kp - K)))
    out = pl.pallas_call(
        functools.partial(_mm_act_kernel, act=act),
        out_shape=jax.ShapeDtypeStruct((Mp, Np), out_dtype),
        grid=(Mp // tm,),
        in_specs=[
            pl.BlockSpec((tm, Kp), lambda i: (i, 0)),
            pl.BlockSpec((Kp, Np), lambda i: (0, 0)),
            pl.BlockSpec((1, Np), lambda i: (0, 0)),
        ],
        out_specs=pl.BlockSpec((tm, Np), lambda i: (i, 0)),
        compiler_params=pltpu.CompilerParams(
            dimension_semantics=("parallel",),
            vmem_limit_bytes=_vmem_budget(
                jax.ShapeDtypeStruct((tm, Kp), jnp.bfloat16),
                jax.ShapeDtypeStruct((tm, Np), out_dtype),
                jax.ShapeDtypeStruct((Kp, Np), jnp.bfloat16),
            ),
        ),
    )(xb, w, b)
    if n_out is None and Mp == M:
        return out
    return out[:M, : (Np if n_out is None else n_out)]


def _enc0_kernel(x_ref, w_ref, b_ref, o_ref, *, tb):
    wins = []
    for i in range(7):
        slab = x_ref[0, 0, i : i + tb]
        wins.append(jnp.concatenate([slab[:, 0:16, :], slab[:, 1:17, :]], axis=2))
    p = jnp.concatenate(wins, axis=2).reshape(tb * 16, 7 * 256)
    y = jnp.dot(p, w_ref[...], preferred_element_type=jnp.float32)
    y = jnp.maximum(y + b_ref[...], 0.0)
    o_ref[0, 0] = y.reshape(tb, 16, 2048).astype(o_ref.dtype)


def _enc0_conv(x, w, b, tb=32):
    N = x.shape[0]
    xp = jnp.pad(x, ((0, 0), (0, 0), (0, 26), (0, 0))).reshape(N, 518, 17, 128)
    nb = 512 // tb
    xb = _bands(xp, tb, tb + 6, nb)
    w7 = w[:196, :64].astype(jnp.float32).reshape(7, 7, 4, 64)
    wg = jnp.zeros((7, 64, 4, 32, 64), jnp.float32)
    for t in range(32):
        wg = wg.at[:, t : t + 7, :, t, :].set(w7)
    wg = wg.reshape(1792, 2048).astype(jnp.bfloat16)
    bg = jnp.tile(b[0, :64], 32).reshape(1, 2048)
    out = pl.pallas_call(
        functools.partial(_enc0_kernel, tb=tb),
        out_shape=jax.ShapeDtypeStruct((N, nb, tb, 16, 2048), jnp.bfloat16),
        grid=(N, nb),
        in_specs=[
            pl.BlockSpec((1, 1, tb + 6, 17, 128), lambda n, i: (n, i, 0, 0, 0)),
            pl.BlockSpec((1792, 2048), lambda n, i: (0, 0)),
            pl.BlockSpec((1, 2048), lambda n, i: (0, 0)),
        ],
        out_specs=pl.BlockSpec((1, 1, tb, 16, 2048), lambda n, i: (n, i, 0, 0, 0)),
        compiler_params=pltpu.CompilerParams(
            dimension_semantics=("parallel", "parallel"),
            vmem_limit_bytes=48 * (1 << 20),
        ),
    )(xb, wg, bg)
    return out.reshape(N, 512, 512, 64)


def _bands(x, row0_stride, rows_in, nb):
    return jnp.stack([x[:, row0_stride * b : row0_stride * b + rows_in]
                      for b in range(nb)], axis=1)


def _conv_s2(x, w, b, co, tb, ck):
    N, H, W, ci = x.shape
    wo = W // 2
    nb = (H // 2) // tb
    xp = jnp.pad(x, ((0, 0), (1, 1), (1, 1), (0, 0)))
    xb = _bands(xp, 2 * tb, 2 * tb + 2, nb)
    w9 = w[: 9 * ci].reshape(9, ci, co)
    out = pl.pallas_call(
        functools.partial(_s2_kernel, ci=ci, co=co, wo=wo, tb=tb, ck=ck),
        out_shape=jax.ShapeDtypeStruct((N, nb, tb, wo, co), jnp.bfloat16),
        grid=(N, nb),
        in_specs=[
            pl.BlockSpec((1, 1, 2 * tb + 2, 2 * wo + 2, ci),
                         lambda n, i: (n, i, 0, 0, 0)),
            pl.BlockSpec((9, ci, co), lambda n, i: (0, 0, 0)),
            pl.BlockSpec((1, co), lambda n, i: (0, 0)),
        ],
        out_specs=pl.BlockSpec((1, 1, tb, wo, co), lambda n, i: (n, i, 0, 0, 0)),
        compiler_params=pltpu.CompilerParams(
            dimension_semantics=("parallel", "parallel"),
            vmem_limit_bytes=48 * (1 << 20),
        ),
    )(xb, w9, b[:, :co])
    return out.reshape(N, H // 2, W // 2, co)


def _ffc_block(x, w, b, tb=16, ck=4):
    N, H, W, dim = x.shape
    nb = H // tb
    xlp = jnp.pad(x[..., :64], ((0, 0), (1, 1), (1, 1), (0, 0)))
    xlb = _bands(xlp, tb, tb + 2, nb)
    xrb = x.reshape(N, nb, tb, W, dim)
    wl = w[:576, :256].reshape(9, 64, 256)
    wg = (w[576:768, 256:].astype(jnp.float32)
          + w[768:960, 256:].astype(jnp.float32)).astype(jnp.bfloat16)
    out = pl.pallas_call(
        functools.partial(_ffc_fused_kernel, tb=tb, ck=ck, wi=W),
        out_shape=jax.ShapeDtypeStruct((N, nb, tb, W, dim), jnp.bfloat16),
        grid=(N, nb),
        in_specs=[
            pl.BlockSpec((1, 1, tb + 2, W + 2, 64), lambda n, i: (n, i, 0, 0, 0)),
            pl.BlockSpec((1, 1, tb, W, dim), lambda n, i: (n, i, 0, 0, 0)),
            pl.BlockSpec((9, 64, 256), lambda n, i: (0, 0, 0)),
            pl.BlockSpec((192, 256), lambda n, i: (0, 0)),
            pl.BlockSpec((1, 512), lambda n, i: (0, 0)),
        ],
        out_specs=pl.BlockSpec((1, 1, tb, W, dim), lambda n, i: (n, i, 0, 0, 0)),
        compiler_params=pltpu.CompilerParams(
            dimension_semantics=("parallel", "parallel"),
            vmem_limit_bytes=48 * (1 << 20),
        ),
    )(xlb, xrb, wl, wg, b)
    return out.reshape(N, H, W, dim)


def _conv_up_fused(x, w, b, co, tb, ck):
    N, H, W, ci = x.shape
    nb = H // tb
    xp = jnp.pad(x, ((0, 0), (0, 1), (0, 1), (0, 0)))
    xb = _bands(xp, tb, tb + 1, nb)
    w4 = w[: 4 * ci]
    out = pl.pallas_call(
        functools.partial(_up_kernel, ci=ci, co=co, wo=W, tb=tb, ck=ck),
        out_shape=jax.ShapeDtypeStruct((N, nb, tb, 2, W, 2 * co), jnp.bfloat16),
        grid=(N, nb),
        in_specs=[
            pl.BlockSpec((1, 1, tb + 1, W + 1, ci), lambda n, i: (n, i, 0, 0, 0)),
            pl.BlockSpec((4 * ci, 4 * co), lambda n, i: (0, 0)),
            pl.BlockSpec((1, 4 * co), lambda n, i: (0, 0)),
        ],
        out_specs=pl.BlockSpec((1, 1, tb, 2, W, 2 * co),
                               lambda n, i: (n, i, 0, 0, 0, 0)),
        compiler_params=pltpu.CompilerParams(
            dimension_semantics=("parallel", "parallel"),
            vmem_limit_bytes=48 * (1 << 20),
        ),
    )(xb, w4, b[:, : 4 * co])
    return out.reshape(N, 2 * H, 2 * W, co)


def _im2col(x, k, stride):
    N, H, W, C = x.shape
    Ho = (H - k) // stride + 1
    Wo = (W - k) // stride + 1
    cols = [
        x[:, i : i + stride * (Ho - 1) + 1 : stride,
          j : j + stride * (Wo - 1) + 1 : stride, :]
        for i in range(k)
        for j in range(k)
    ]
    patches = jnp.stack(cols, axis=3).reshape(N * Ho * Wo, k * k * C)
    return patches, Ho, Wo


def _conv(x, w, b, cout, k, stride, pad, act, tm, out_dtype=jnp.bfloat16):
    if pad:
        x = jnp.pad(x, ((0, 0), (pad, pad), (pad, pad), (0, 0)))
    cols, Ho, Wo = _im2col(x, k, stride)
    out = _mm(cols, w, b, act, tm, out_dtype, n_out=cout)
    return out.reshape(x.shape[0], Ho, Wo, cout)


_GH, _GW = 4, 8
_HB = 8


def _head_weight(dec2_w, dec2_b):
    w4 = dec2_w[:3136, :3].astype(jnp.float32).reshape(7, 7, 64, 3)
    wg = jnp.zeros((_GH + 6, _GW + 6, 64, _GH, _GW, 3), jnp.float32)
    for dh in range(_GH):
        for dw in range(_GW):
            wg = wg.at[dh : dh + 7, dw : dw + 7, :, dh, dw, :].set(w4)
    wg = wg.reshape(_GH + 6, (_GW + 6) * 64, _GH * _GW * 3)
    wg = jnp.pad(wg, ((0, 0), (0, 0), (0, 128 - _GH * _GW * 3)))
    bias = jnp.tile(dec2_b[0, :3], _GH * _GW)
    bias = jnp.pad(bias, (0, 128 - _GH * _GW * 3)).reshape(1, -1)
    return wg.astype(jnp.bfloat16), bias.astype(jnp.float32)


def _head_kernel(x_ref, w_ref, b_ref, o_ref):
    xb = x_ref[0, 0].reshape(10, 4, 65, 512)
    acc = None
    for i2 in range(10):
        q, r = divmod(i2, 4)
        slab = xb[q : q + _HB, r]
        win = jnp.concatenate([slab[:, 0:64, :], slab[:, 1:65, :]], axis=2)
        pw = win[:, :, :896].reshape(_HB * 64, 896)
        d = jnp.dot(pw, w_ref[i2], preferred_element_type=jnp.float32)
        acc = d if acc is None else acc + d
    y = jax.nn.sigmoid(acc + b_ref[...])
    o_ref[0, 0] = y.reshape(_HB, 64, 128)


def _head_conv(x, wg, bg):
    N, H, W, _ = x.shape
    xp = jnp.pad(x, ((0, 0), (3, 3), (3, 3), (0, 0)), mode="reflect")
    xp = jnp.pad(xp, ((0, 0), (0, 2), (0, 2), (0, 0)))
    xr = xp.reshape(N, H + 8, (W + 8) // 8, 512)
    nb = H // (_GH * _HB)
    bands = jnp.stack([xr[:, 32 * b : 32 * b + 40] for b in range(nb)], axis=1)
    y = pl.pallas_call(
        _head_kernel,
        out_shape=jax.ShapeDtypeStruct((N, nb, _HB, 64, 128), jnp.float32),
        grid=(N, nb),
        in_specs=[
            pl.BlockSpec((1, 1, 40, 65, 512), lambda n, b: (n, b, 0, 0, 0)),
            pl.BlockSpec((10, 896, 128), lambda n, b: (0, 0, 0)),
            pl.BlockSpec((1, 128), lambda n, b: (0, 0)),
        ],
        out_specs=pl.BlockSpec((1, 1, _HB, 64, 128), lambda n, b: (n, b, 0, 0, 0)),
        compiler_params=pltpu.CompilerParams(
            dimension_semantics=("parallel", "parallel"),
            vmem_limit_bytes=48 * (1 << 20),
        ),
    )(bands, wg, bg)
    y = y[..., :96].reshape(N, nb, _HB, 64, _GH, _GW, 3)
    y = y.transpose(0, 1, 2, 4, 3, 5, 6)
    return y.reshape(N, H, W, 3)


def kernel(image, mask, enc0_w, enc0_b, enc1_w, enc1_b, enc2_w, enc2_b,
           mid0_w, mid0_b, mid1_w, mid1_b, dec0_w, dec0_b, dec1_w, dec1_b,
           dec2_w, dec2_b):
    ngf = 64
    img = jnp.transpose(image, (0, 2, 3, 1))
    msk = jnp.transpose(mask, (0, 2, 3, 1))
    keep = 1.0 - msk
    x = jnp.concatenate([img * keep, msk], axis=-1).astype(jnp.bfloat16)

    x = jnp.pad(x, ((0, 0), (3, 3), (3, 3), (0, 0)), mode="reflect")
    x = _enc0_conv(x, enc0_w, enc0_b)
    x = _conv_s2(x, enc1_w, enc1_b, ngf * 2, tb=8, ck=4)
    x = _conv_s2(x, enc2_w, enc2_b, ngf * 4, tb=8, ck=8)

    x = _ffc_block(x, mid0_w, mid0_b)
    x = _ffc_block(x, mid1_w, mid1_b)

    x = _conv_up_fused(x, dec0_w, dec0_b, ngf * 2, tb=16, ck=4)
    x = _conv_up_fused(x, dec1_w, dec1_b, ngf, tb=16, ck=4)
    wg, bg = _head_weight(dec2_w, dec2_b)
    y = _head_conv(x, wg, bg)

    out = img * keep + y * msk
    return {"inpainted": jnp.transpose(out, (0, 3, 1, 2))}


# speedup vs baseline: 1.2549x; 1.0004x over previous
import functools

import jax
import jax.numpy as jnp
from jax.experimental import pallas as pl
from jax.experimental.pallas import tpu as pltpu


def _ru(x, m):
    return (x + m - 1) // m * m


def _s2_kernel(x_ref, w_ref, b_ref, o_ref, *, ci, co, wo, tb, ck):
    xx = x_ref[0, 0].reshape(tb + 1, 2, wo + 1, 2, ci)
    for c0 in range(0, tb, ck):
        acc = None
        for i in range(3):
            qi, si = divmod(i, 2)
            for j in range(3):
                kj, pj = divmod(j, 2)
                slab = xx[c0 + qi : c0 + qi + ck, si, kj : kj + wo, pj, :]
                d = jnp.dot(slab.reshape(ck * wo, ci), w_ref[3 * i + j],
                            preferred_element_type=jnp.float32)
                acc = d if acc is None else acc + d
        y = jnp.maximum(acc + b_ref[...], 0.0)
        o_ref[0, 0, c0 : c0 + ck] = y.reshape(ck, wo, co).astype(o_ref.dtype)


def _ffc_fused_kernel(xl_ref, x_ref, wl_ref, wg_ref, b_ref, o_ref, *, tb, ck, wi):
    for c0 in range(0, tb, ck):
        xc = x_ref[0, 0, c0 : c0 + ck].reshape(ck * wi, 256)
        accg = jnp.dot(xc[:, 64:], wg_ref[...], preferred_element_type=jnp.float32)
        y1 = jnp.maximum(accg + b_ref[:, 256:], 0.0)
        accl = None
        for i in range(3):
            for j in range(3):
                slab = xl_ref[0, 0, c0 + i : c0 + i + ck, j : j + wi, :]
                d = jnp.dot(slab.reshape(ck * wi, 64), wl_ref[3 * i + j],
                            preferred_element_type=jnp.float32)
                accl = d if accl is None else accl + d
        y0 = jnp.maximum(accl + b_ref[:, :256], 0.0)
        o = y0 + y1 + xc.astype(jnp.float32)
        o_ref[0, 0, c0 : c0 + ck] = o.reshape(ck, wi, 256).astype(o_ref.dtype)


def _up_kernel(x_ref, w_ref, b_ref, o_ref, *, ci, co, wo, tb, ck):
    for c0 in range(0, tb, ck):
        acc = None
        for t, (i, j) in enumerate(((0, 0), (0, 1), (1, 0), (1, 1))):
            slab = x_ref[0, 0, c0 + i : c0 + i + ck, j : j + wo, :]
            d = jnp.dot(slab.reshape(ck * wo, ci), w_ref[t * ci : (t + 1) * ci],
                        preferred_element_type=jnp.float32)
            acc = d if acc is None else acc + d
        y = jnp.maximum(acc + b_ref[...], 0.0).astype(o_ref.dtype)
        y = y.reshape(ck, wo, 4 * co)
        for a in (0, 1):
            o_ref[0, 0, c0 : c0 + ck, a] = y[:, :, a * 2 * co : (a + 1) * 2 * co]


def _enc0_kernel(x_ref, w_ref, b_ref, o_ref, *, tb):
    wins = []
    for i in range(7):
        slab = x_ref[0, 0, i : i + tb]
        wins.append(jnp.concatenate([slab[:, 0:16, :], slab[:, 1:17, :]], axis=2))
    p = jnp.concatenate(wins, axis=2).reshape(tb * 16, 7 * 256)
    y = jnp.dot(p, w_ref[...], preferred_element_type=jnp.float32)
    y = jnp.maximum(y + b_ref[...], 0.0)
    o_ref[0, 0] = y.reshape(tb, 16, 2048).astype(o_ref.dtype)


def _enc0_conv(x, w, b, tb=32):
    N = x.shape[0]
    xp = jnp.pad(x, ((0, 0), (0, 0), (0, 26), (0, 0))).reshape(N, 518, 17, 128)
    nb = 512 // tb
    xb = _bands(xp, tb, tb + 6, nb)
    w7 = w[:196, :64].astype(jnp.float32).reshape(7, 7, 4, 64)
    wg = jnp.zeros((7, 64, 4, 32, 64), jnp.float32)
    for t in range(32):
        wg = wg.at[:, t : t + 7, :, t, :].set(w7)
    wg = wg.reshape(1792, 2048).astype(jnp.bfloat16)
    bg = jnp.tile(b[0, :64], 32).reshape(1, 2048)
    out = pl.pallas_call(
        functools.partial(_enc0_kernel, tb=tb),
        out_shape=jax.ShapeDtypeStruct((N, nb, tb, 16, 2048), jnp.bfloat16),
        grid=(N, nb),
        in_specs=[
            pl.BlockSpec((1, 1, tb + 6, 17, 128), lambda n, i: (n, i, 0, 0, 0)),
            pl.BlockSpec((1792, 2048), lambda n, i: (0, 0)),
            pl.BlockSpec((1, 2048), lambda n, i: (0, 0)),
        ],
        out_specs=pl.BlockSpec((1, 1, tb, 16, 2048), lambda n, i: (n, i, 0, 0, 0)),
        compiler_params=pltpu.CompilerParams(
            dimension_semantics=("parallel", "parallel"),
            vmem_limit_bytes=48 * (1 << 20),
        ),
    )(xb, wg, bg)
    return out.reshape(N, 512, 512, 64)


def _bands(x, row0_stride, rows_in, nb):
    return jnp.stack([x[:, row0_stride * b : row0_stride * b + rows_in]
                      for b in range(nb)], axis=1)


def _conv_s2(x, w, b, co, tb, ck):
    N, H, W, ci = x.shape
    wo = W // 2
    nb = (H // 2) // tb
    xp = jnp.pad(x, ((0, 0), (1, 1), (1, 1), (0, 0)))
    xb = _bands(xp, 2 * tb, 2 * tb + 2, nb)
    w9 = w[: 9 * ci].reshape(9, ci, co)
    out = pl.pallas_call(
        functools.partial(_s2_kernel, ci=ci, co=co, wo=wo, tb=tb, ck=ck),
        out_shape=jax.ShapeDtypeStruct((N, nb, tb, wo, co), jnp.bfloat16),
        grid=(N, nb),
        in_specs=[
            pl.BlockSpec((1, 1, 2 * tb + 2, 2 * wo + 2, ci),
                         lambda n, i: (n, i, 0, 0, 0)),
            pl.BlockSpec((9, ci, co), lambda n, i: (0, 0, 0)),
            pl.BlockSpec((1, co), lambda n, i: (0, 0)),
        ],
        out_specs=pl.BlockSpec((1, 1, tb, wo, co), lambda n, i: (n, i, 0, 0, 0)),
        compiler_params=pltpu.CompilerParams(
            dimension_semantics=("parallel", "parallel"),
            vmem_limit_bytes=48 * (1 << 20),
        ),
    )(xb, w9, b[:, :co])
    return out.reshape(N, H // 2, W // 2, co)


def _ffc_block(x, w, b, tb=16, ck=4):
    N, H, W, dim = x.shape
    nb = H // tb
    xlp = jnp.pad(x[..., :64], ((0, 0), (1, 1), (1, 1), (0, 0)))
    xlb = _bands(xlp, tb, tb + 2, nb)
    xrb = x.reshape(N, nb, tb, W, dim)
    wl = w[:576, :256].reshape(9, 64, 256)
    wg = (w[576:768, 256:].astype(jnp.float32)
          + w[768:960, 256:].astype(jnp.float32)).astype(jnp.bfloat16)
    out = pl.pallas_call(
        functools.partial(_ffc_fused_kernel, tb=tb, ck=ck, wi=W),
        out_shape=jax.ShapeDtypeStruct((N, nb, tb, W, dim), jnp.bfloat16),
        grid=(N, nb),
        in_specs=[
            pl.BlockSpec((1, 1, tb + 2, W + 2, 64), lambda n, i: (n, i, 0, 0, 0)),
            pl.BlockSpec((1, 1, tb, W, dim), lambda n, i: (n, i, 0, 0, 0)),
            pl.BlockSpec((9, 64, 256), lambda n, i: (0, 0, 0)),
            pl.BlockSpec((192, 256), lambda n, i: (0, 0)),
            pl.BlockSpec((1, 512), lambda n, i: (0, 0)),
        ],
        out_specs=pl.BlockSpec((1, 1, tb, W, dim), lambda n, i: (n, i, 0, 0, 0)),
        compiler_params=pltpu.CompilerParams(
            dimension_semantics=("parallel", "parallel"),
            vmem_limit_bytes=48 * (1 << 20),
        ),
    )(xlb, xrb, wl, wg, b)
    return out.reshape(N, H, W, dim)


def _conv_up_fused(x, w, b, co, tb, ck):
    N, H, W, ci = x.shape
    nb = H // tb
    xp = jnp.pad(x, ((0, 0), (0, 1), (0, 1), (0, 0)))
    xb = _bands(xp, tb, tb + 1, nb)
    w4 = w[: 4 * ci]
    out = pl.pallas_call(
        functools.partial(_up_kernel, ci=ci, co=co, wo=W, tb=tb, ck=ck),
        out_shape=jax.ShapeDtypeStruct((N, nb, tb, 2, W, 2 * co), jnp.bfloat16),
        grid=(N, nb),
        in_specs=[
            pl.BlockSpec((1, 1, tb + 1, W + 1, ci), lambda n, i: (n, i, 0, 0, 0)),
            pl.BlockSpec((4 * ci, 4 * co), lambda n, i: (0, 0)),
            pl.BlockSpec((1, 4 * co), lambda n, i: (0, 0)),
        ],
        out_specs=pl.BlockSpec((1, 1, tb, 2, W, 2 * co),
                               lambda n, i: (n, i, 0, 0, 0, 0)),
        compiler_params=pltpu.CompilerParams(
            dimension_semantics=("parallel", "parallel"),
            vmem_limit_bytes=48 * (1 << 20),
        ),
    )(xb, w4, b[:, : 4 * co])
    return out.reshape(N, 2 * H, 2 * W, co)


_GH, _GW = 4, 8
_HB = 8


def _head_weight(dec2_w, dec2_b):
    w4 = dec2_w[:3136, :3].astype(jnp.float32).reshape(7, 7, 64, 3)
    wg = jnp.zeros((_GH + 6, _GW + 6, 64, _GH, _GW, 3), jnp.float32)
    for dh in range(_GH):
        for dw in range(_GW):
            wg = wg.at[dh : dh + 7, dw : dw + 7, :, dh, dw, :].set(w4)
    wg = wg.reshape(_GH + 6, (_GW + 6) * 64, _GH * _GW * 3)
    wg = jnp.pad(wg, ((0, 0), (0, 0), (0, 128 - _GH * _GW * 3)))
    bias = jnp.tile(dec2_b[0, :3], _GH * _GW)
    bias = jnp.pad(bias, (0, 128 - _GH * _GW * 3)).reshape(1, -1)
    return wg.astype(jnp.bfloat16), bias.astype(jnp.float32)


def _head_kernel(x_ref, w_ref, b_ref, o_ref):
    xb = x_ref[0, 0].reshape(10, 4, 65, 512)
    acc = None
    for i2 in range(10):
        q, r = divmod(i2, 4)
        slab = xb[q : q + _HB, r]
        win = jnp.concatenate([slab[:, 0:64, :], slab[:, 1:65, :]], axis=2)
        pw = win[:, :, :896].reshape(_HB * 64, 896)
        d = jnp.dot(pw, w_ref[i2], preferred_element_type=jnp.float32)
        acc = d if acc is None else acc + d
    y = jax.nn.sigmoid(acc + b_ref[...])
    o_ref[0, 0] = y.reshape(_HB, 64, 128)


def _head_conv(x, wg, bg):
    N, H, W, _ = x.shape
    xp = jnp.pad(x, ((0, 0), (3, 3), (3, 3), (0, 0)), mode="reflect")
    xp = jnp.pad(xp, ((0, 0), (0, 2), (0, 2), (0, 0)))
    xr = xp.reshape(N, H + 8, (W + 8) // 8, 512)
    nb = H // (_GH * _HB)
    bands = jnp.stack([xr[:, 32 * b : 32 * b + 40] for b in range(nb)], axis=1)
    y = pl.pallas_call(
        _head_kernel,
        out_shape=jax.ShapeDtypeStruct((N, nb, _HB, 64, 128), jnp.float32),
        grid=(N, nb),
        in_specs=[
            pl.BlockSpec((1, 1, 40, 65, 512), lambda n, b: (n, b, 0, 0, 0)),
            pl.BlockSpec((10, 896, 128), lambda n, b: (0, 0, 0)),
            pl.BlockSpec((1, 128), lambda n, b: (0, 0)),
        ],
        out_specs=pl.BlockSpec((1, 1, _HB, 64, 128), lambda n, b: (n, b, 0, 0, 0)),
        compiler_params=pltpu.CompilerParams(
            dimension_semantics=("parallel", "parallel"),
            vmem_limit_bytes=48 * (1 << 20),
        ),
    )(bands, wg, bg)
    y = y[..., :96].reshape(N, nb, _HB, 64, _GH, _GW, 3)
    y = y.transpose(0, 1, 2, 4, 3, 5, 6)
    return y.reshape(N, H, W, 3)


def kernel(image, mask, enc0_w, enc0_b, enc1_w, enc1_b, enc2_w, enc2_b,
           mid0_w, mid0_b, mid1_w, mid1_b, dec0_w, dec0_b, dec1_w, dec1_b,
           dec2_w, dec2_b):
    ngf = 64
    img = jnp.transpose(image, (0, 2, 3, 1))
    msk = jnp.transpose(mask, (0, 2, 3, 1))
    keep = 1.0 - msk
    x = jnp.concatenate([img * keep, msk], axis=-1).astype(jnp.bfloat16)

    x = jnp.pad(x, ((0, 0), (3, 3), (3, 3), (0, 0)), mode="reflect")
    x = _enc0_conv(x, enc0_w, enc0_b)
    x = _conv_s2(x, enc1_w, enc1_b, ngf * 2, tb=8, ck=4)
    x = _conv_s2(x, enc2_w, enc2_b, ngf * 4, tb=8, ck=8)

    x = _ffc_block(x, mid0_w, mid0_b)
    x = _ffc_block(x, mid1_w, mid1_b)

    x = _conv_up_fused(x, dec0_w, dec0_b, ngf * 2, tb=16, ck=4)
    x = _conv_up_fused(x, dec1_w, dec1_b, ngf, tb=16, ck=4)
    wg, bg = _head_weight(dec2_w, dec2_b)
    y = _head_conv(x, wg, bg)

    out = img * keep + y * msk
    return {"inpainted": jnp.transpose(out, (0, 3, 1, 2))}
